# baseline (speedup 1.0000x reference)
import functools

import jax
import jax.numpy as jnp
from jax import lax
from jax.experimental import pallas as pl
from jax.experimental.pallas import tpu as pltpu
from jax.experimental.pallas import tpu_sc as plsc

_NUM_CORES = 2
_NUM_SUBCORES = 16
_NUM_WORKERS = _NUM_CORES * _NUM_SUBCORES

_CHUNK = 1024


@functools.partial(jax.jit, static_argnames=("b_total", "d"))
def _sc_gather(table, idx_flat, *, b_total, d):
  b_per_w = b_total // _NUM_WORKERS
  n_chunks = b_per_w // _CHUNK
  mesh = plsc.VectorSubcoreMesh(
      core_axis_name="c", subcore_axis_name="s",
      num_cores=_NUM_CORES, num_subcores=_NUM_SUBCORES)

  @functools.partial(
      pl.kernel,
      mesh=mesh,
      out_type=jax.ShapeDtypeStruct((b_total, d), jnp.float32),
      scratch_types=[
          pltpu.VMEM((_CHUNK,), jnp.int32),
          pltpu.VMEM((_CHUNK, d), jnp.float32),
          pltpu.SemaphoreType.DMA,
      ],
      compiler_params=pltpu.CompilerParams(use_tc_tiling_on_sc=False),
  )
  def k(table_hbm, idx_hbm, out_hbm, idx_v, rows_v, sem):
    wid = lax.axis_index("s") * _NUM_CORES + lax.axis_index("c")
    base = wid * b_per_w

    def body(i, _):
      off = base + i * _CHUNK
      pltpu.sync_copy(idx_hbm.at[pl.ds(off, _CHUNK)], idx_v)
      pltpu.async_copy(table_hbm.at[idx_v], rows_v, sem).wait()
      pltpu.sync_copy(rows_v, out_hbm.at[pl.ds(off, _CHUNK)])
      return 0

    lax.fori_loop(0, n_chunks, body, 0)

  return k(table, idx_flat)


def kernel(input_ids, table):
  b_total = input_ids.size
  d = table.shape[1]
  idx_flat = input_ids.reshape(-1)
  out = _sc_gather(table, idx_flat, b_total=b_total, d=d)
  return out.reshape(*input_ids.shape, d)


# speedup vs baseline: 1.0066x; 1.0066x over previous
import functools

import jax
import jax.numpy as jnp
from jax import lax
from jax.experimental import pallas as pl
from jax.experimental.pallas import tpu as pltpu
from jax.experimental.pallas import tpu_sc as plsc

_NUM_CORES = 2
_NUM_SUBCORES = 16
_NUM_WORKERS = _NUM_CORES * _NUM_SUBCORES

_CHUNK = 800


@functools.partial(jax.jit, static_argnames=("b_total", "d"))
def _sc_gather(table, idx_flat, *, b_total, d):
  b_per_w = b_total // _NUM_WORKERS
  n_chunks = b_per_w // _CHUNK
  n_groups = n_chunks // 2
  mesh = plsc.VectorSubcoreMesh(
      core_axis_name="c", subcore_axis_name="s",
      num_cores=_NUM_CORES, num_subcores=_NUM_SUBCORES)

  @functools.partial(
      pl.kernel,
      mesh=mesh,
      out_type=jax.ShapeDtypeStruct((b_total, d), jnp.float32),
      scratch_types=[
          pltpu.VMEM((n_chunks + 1, _CHUNK), jnp.int32),
          pltpu.VMEM((_CHUNK, d), jnp.float32),
          pltpu.VMEM((_CHUNK, d), jnp.float32),
          pltpu.SemaphoreType.DMA,
          pltpu.SemaphoreType.DMA,
          pltpu.SemaphoreType.DMA,
          pltpu.SemaphoreType.DMA,
      ],
      compiler_params=pltpu.CompilerParams(use_tc_tiling_on_sc=False),
  )
  def k(table_hbm, idx_hbm, out_hbm, idx_v, rows0, rows1, g0, g1, o0, o1):
    wid = lax.axis_index("s") * _NUM_CORES + lax.axis_index("c")
    base = wid * b_per_w

    row0 = wid * n_chunks
    pltpu.sync_copy(idx_hbm.at[pl.ds(row0, n_chunks)],
                    idx_v.at[pl.ds(0, n_chunks)])
    pltpu.sync_copy(idx_hbm.at[pl.ds(row0, 1)],
                    idx_v.at[pl.ds(n_chunks, 1)])

    def g_start(i, buf, sem):
      pltpu.async_copy(table_hbm.at[idx_v.at[i]], buf, sem)

    def g_wait(i, buf, sem):
      pltpu.make_async_copy(table_hbm.at[idx_v.at[i]], buf, sem).wait()

    def o_start(i, buf, sem):
      pltpu.async_copy(buf, out_hbm.at[pl.ds(base + i * _CHUNK, _CHUNK)], sem)

    def o_wait(i, buf, sem):
      pltpu.make_async_copy(
          buf, out_hbm.at[pl.ds(base + i * _CHUNK, _CHUNK)], sem).wait()

    g_start(0, rows0, g0)
    g_wait(0, rows0, g0)
    o_start(0, rows0, o0)
    g_start(1, rows1, g1)
    g_wait(1, rows1, g1)
    o_start(1, rows1, o1)
    o_wait(0, rows0, o0)
    g_start(2, rows0, g0)

    def body(g, _):
      i = 2 * g
      g_wait(i, rows0, g0)
      o_start(i, rows0, o0)
      o_wait(i - 1, rows1, o1)
      g_start(i + 1, rows1, g1)
      g_wait(i + 1, rows1, g1)
      o_start(i + 1, rows1, o1)
      o_wait(i, rows0, o0)
      g_start(i + 2, rows0, g0)
      return 0

    lax.fori_loop(1, n_groups, body, 0)

    o_wait(n_chunks - 1, rows1, o1)
    g_wait(n_chunks, rows0, g0)

  return k(table, idx_flat)


def kernel(input_ids, table):
  b_total = input_ids.size
  d = table.shape[1]
  idx_2d = input_ids.reshape(b_total // _CHUNK, _CHUNK)
  out = _sc_gather(table, idx_2d, b_total=b_total, d=d)
  return out.reshape(*input_ids.shape, d)
